# baseline (speedup 1.0000x reference)
import dataclasses
import functools

import jax
import jax.numpy as jnp
from jax import lax
from jax.experimental import pallas as pl
from jax.experimental.pallas import tpu as pltpu
from jax.experimental.pallas import tpu_sc as plsc

C = 10000
NW = 32
L = 16
BLK = 2000

_MESH = plsc.VectorSubcoreMesh(
    core_axis_name="c", subcore_axis_name="s", num_cores=2, num_subcores=16)

_SC_PARAMS = pltpu.CompilerParams()
if "needs_layout_passes" in pltpu.CompilerParams.__dataclass_fields__:
    _SC_PARAMS = dataclasses.replace(_SC_PARAMS, needs_layout_passes=False)
if "use_tc_tiling_on_sc" in pltpu.CompilerParams.__dataclass_fields__:
    _SC_PARAMS = dataclasses.replace(_SC_PARAMS, use_tc_tiling_on_sc=False)


def _worker_id():
    return lax.axis_index("s") * 2 + lax.axis_index("c")


def _k1_body(x_hbm, y_hbm, z_hbm, comp_hbm, out_hbm,
             cnt_a, sx_a, sy_a, sz_a,
             xb, yb, zb, cbuf, xb2, yb2, zb2, cbuf2, sem0, sem1):
    n = comp_hbm.shape[0]
    chunk = n // NW
    nblk = chunk // BLK
    wid = _worker_id()
    base = wid * chunk
    lanes = lax.iota(jnp.int32, L)
    zeros = jnp.zeros((L,), jnp.float32)
    ones = jnp.ones((L,), jnp.float32)

    @pl.loop(0, C, step=L)
    def _zero(i):
        cnt_a[pl.ds(i, L)] = zeros
        sx_a[pl.ds(i, L)] = zeros
        sy_a[pl.ds(i, L)] = zeros
        sz_a[pl.ds(i, L)] = zeros

    lane0 = lanes == 0

    def flush(cur, vc, vx, vy, vz):
        bcur = jnp.broadcast_to(cur, (L,))
        plsc.addupdate_scatter(
            cnt_a, [bcur], jnp.broadcast_to(jnp.sum(vc), (L,)), mask=lane0)
        plsc.addupdate_scatter(
            sx_a, [bcur], jnp.broadcast_to(jnp.sum(vx), (L,)), mask=lane0)
        plsc.addupdate_scatter(
            sy_a, [bcur], jnp.broadcast_to(jnp.sum(vy), (L,)), mask=lane0)
        plsc.addupdate_scatter(
            sz_a, [bcur], jnp.broadcast_to(jnp.sum(vz), (L,)), mask=lane0)

    def dma4(bufs, sem, b, start):
        off = base + b * BLK
        cps = (
            pltpu.make_async_copy(x_hbm.at[pl.ds(off, BLK)], bufs[0], sem),
            pltpu.make_async_copy(y_hbm.at[pl.ds(off, BLK)], bufs[1], sem),
            pltpu.make_async_copy(z_hbm.at[pl.ds(off, BLK)], bufs[2], sem),
            pltpu.make_async_copy(comp_hbm.at[pl.ds(off, BLK)], bufs[3], sem),
        )
        for cp in cps:
            if start:
                cp.start()
            else:
                cp.wait()

    def block_compute(xb, yb, zb, cbuf):

        @pl.loop(0, BLK, step=L)
        def _step(i):
            x = xb[pl.ds(i, L)]
            y = yb[pl.ds(i, L)]
            z = zb[pl.ds(i, L)]
            idx = cbuf[pl.ds(i, L)]
            plsc.addupdate_scatter(cnt_a, [idx], ones)
            plsc.addupdate_scatter(sx_a, [idx], x)
            plsc.addupdate_scatter(sy_a, [idx], y)
            plsc.addupdate_scatter(sz_a, [idx], z)

    b0 = (xb, yb, zb, cbuf)
    b1 = (xb2, yb2, zb2, cbuf2)
    dma4(b0, sem0, 0, True)

    @pl.loop(0, nblk // 2)
    def _pair(t):
        dma4(b0, sem0, 2 * t, False)
        dma4(b1, sem1, 2 * t + 1, True)
        block_compute(*b0)
        dma4(b1, sem1, 2 * t + 1, False)
        dma4(b0, sem0, 2 * t + 2, True)
        block_compute(*b1)

    dma4(b0, sem0, nblk - 1, False)
    block_compute(*b0)

    pltpu.sync_copy(cnt_a, out_hbm.at[wid, 0])
    pltpu.sync_copy(sx_a, out_hbm.at[wid, 1])
    pltpu.sync_copy(sy_a, out_hbm.at[wid, 2])
    pltpu.sync_copy(sz_a, out_hbm.at[wid, 3])


def _k2_body(part_ref, ctr_ref):
    s = jnp.sum(part_ref[...], axis=0)
    deg = s[0:1, :]
    valid = deg > 0.5
    degs = jnp.where(valid, deg, 1.0)
    xyz = s[1:4, :]
    ctr = jnp.where(valid, xyz / degs, xyz)
    ctr_ref[...] = jnp.concatenate([ctr, deg], axis=0)


def _k3_body(x_hbm, y_hbm, z_hbm, comp_hbm, ctr_hbm, out_hbm,
             cxv, cyv, czv, dmx,
             xb, yb, zb, cbuf, xb2, yb2, zb2, cbuf2, sbuf, sem0, sem1):
    n = comp_hbm.shape[0]
    chunk = n // NW
    nblk = chunk // BLK
    wid = _worker_id()
    base = wid * chunk
    lanes = lax.iota(jnp.int32, L)
    lane0 = lanes == 0
    zeros = jnp.zeros((L,), jnp.float32)

    pltpu.sync_copy(ctr_hbm.at[0], cxv)
    pltpu.sync_copy(ctr_hbm.at[1], cyv)
    pltpu.sync_copy(ctr_hbm.at[2], czv)

    @pl.loop(0, C, step=L)
    def _zero(i):
        dmx[pl.ds(i, L)] = zeros

    cbuf[pl.ds(BLK, L)] = jnp.full((L,), -1, jnp.int32)
    cbuf2[pl.ds(BLK, L)] = jnp.full((L,), -1, jnp.int32)

    def dma4(bufs, sem, b, start):
        off = base + b * BLK
        cps = (
            pltpu.make_async_copy(x_hbm.at[pl.ds(off, BLK)], bufs[0], sem),
            pltpu.make_async_copy(y_hbm.at[pl.ds(off, BLK)], bufs[1], sem),
            pltpu.make_async_copy(z_hbm.at[pl.ds(off, BLK)], bufs[2], sem),
            pltpu.make_async_copy(
                comp_hbm.at[pl.ds(off, BLK)], bufs[3].at[pl.ds(0, BLK)], sem),
        )
        for cp in cps:
            if start:
                cp.start()
            else:
                cp.wait()

    def block_compute(xb, yb, zb, cbuf):

        def step(i_step, carry):
            cur, vmax = carry
            i = i_step * L
            x = xb[pl.ds(i, L)]
            y = yb[pl.ds(i, L)]
            z = zb[pl.ds(i, L)]
            idx = cbuf[pl.ds(i, L)]
            c0 = idx[0]
            c15 = idx[L - 1]
            cx = plsc.load_gather(cxv, [idx])
            cy = plsc.load_gather(cyv, [idx])
            cz = plsc.load_gather(czv, [idx])
            dx = x - cx
            dy = y - cy
            dz = z - cz
            d2 = dx * dx + dy * dy + dz * dz

            def flush():
                bcur = jnp.broadcast_to(cur, (L,))
                m = jnp.broadcast_to(jnp.max(vmax), (L,))
                old = plsc.load_gather(dmx, [bcur], mask=lane0)
                plsc.store_scatter(
                    dmx, [bcur], jnp.maximum(old, m), mask=lane0)

            def uniform_vec():
                def same():
                    return cur, jnp.maximum(vmax, d2)

                def boundary():
                    flush()
                    return c0, d2

                return lax.cond(c0 == cur, same, boundary)

            def mixed_vec():
                flush()
                v = d2
                for s in (1, 2, 4, 8):
                    sbuf[...] = v
                    vs = plsc.load_gather(sbuf, [jnp.maximum(lanes - s, 0)])
                    ids = plsc.load_gather(
                        cbuf, [jnp.maximum(lanes + (i - s), 0)])
                    keep = (lanes >= s) & (idx == ids)
                    v = jnp.where(keep, jnp.maximum(v, vs), v)
                nxt = plsc.load_gather(cbuf, [lanes + (i + 1)])
                endm = (idx != nxt) | (lanes == L - 1)
                old = plsc.load_gather(dmx, [idx], mask=endm)
                plsc.store_scatter(dmx, [idx], jnp.maximum(old, v), mask=endm)
                return c15, zeros

            return lax.cond(c0 == c15, uniform_vec, mixed_vec)

        init = (cbuf[pl.ds(0, L)][0], zeros)
        cur, vmax = lax.fori_loop(0, BLK // L, step, init)
        bcur = jnp.broadcast_to(cur, (L,))
        m = jnp.broadcast_to(jnp.max(vmax), (L,))
        old = plsc.load_gather(dmx, [bcur], mask=lane0)
        plsc.store_scatter(dmx, [bcur], jnp.maximum(old, m), mask=lane0)

    b0 = (xb, yb, zb, cbuf)
    b1 = (xb2, yb2, zb2, cbuf2)
    dma4(b0, sem0, 0, True)

    @pl.loop(0, nblk // 2)
    def _pair(t):
        dma4(b0, sem0, 2 * t, False)
        dma4(b1, sem1, 2 * t + 1, True)
        block_compute(*b0)
        dma4(b1, sem1, 2 * t + 1, False)
        dma4(b0, sem0, 2 * t + 2, True)
        block_compute(*b1)

    dma4(b0, sem0, nblk - 1, False)
    block_compute(*b0)

    pltpu.sync_copy(dmx, out_hbm.at[wid])


def _k4_body(ctr_ref, dmx_ref, out_ref):
    m = jnp.max(dmx_ref[...], axis=0, keepdims=True)
    deg = ctr_ref[3:4, :]
    valid = deg > 0.5
    dia = jnp.where(valid, 2.0 * jnp.sqrt(m + 1e-12), 0.0)
    out_ref[...] = jnp.concatenate([ctr_ref[0:3, :], dia], axis=0)


_k1 = functools.partial(
    pl.kernel,
    out_type=jax.ShapeDtypeStruct((NW, 4, C), jnp.float32),
    mesh=_MESH,
    scratch_types=[
        pltpu.VMEM((C,), jnp.float32),
        pltpu.VMEM((C,), jnp.float32),
        pltpu.VMEM((C,), jnp.float32),
        pltpu.VMEM((C,), jnp.float32),
        pltpu.VMEM((BLK,), jnp.float32),
        pltpu.VMEM((BLK,), jnp.float32),
        pltpu.VMEM((BLK,), jnp.float32),
        pltpu.VMEM((BLK,), jnp.int32),
        pltpu.VMEM((BLK,), jnp.float32),
        pltpu.VMEM((BLK,), jnp.float32),
        pltpu.VMEM((BLK,), jnp.float32),
        pltpu.VMEM((BLK,), jnp.int32),
        pltpu.SemaphoreType.DMA,
        pltpu.SemaphoreType.DMA,
    ],
    compiler_params=_SC_PARAMS,
)(_k1_body)

_k2 = pl.pallas_call(
    _k2_body,
    out_shape=jax.ShapeDtypeStruct((4, C), jnp.float32),
)

_k3 = functools.partial(
    pl.kernel,
    out_type=jax.ShapeDtypeStruct((NW, C), jnp.float32),
    mesh=_MESH,
    scratch_types=[
        pltpu.VMEM((C,), jnp.float32),
        pltpu.VMEM((C,), jnp.float32),
        pltpu.VMEM((C,), jnp.float32),
        pltpu.VMEM((C,), jnp.float32),
        pltpu.VMEM((BLK,), jnp.float32),
        pltpu.VMEM((BLK,), jnp.float32),
        pltpu.VMEM((BLK,), jnp.float32),
        pltpu.VMEM((BLK + L,), jnp.int32),
        pltpu.VMEM((BLK,), jnp.float32),
        pltpu.VMEM((BLK,), jnp.float32),
        pltpu.VMEM((BLK,), jnp.float32),
        pltpu.VMEM((BLK + L,), jnp.int32),
        pltpu.VMEM((L,), jnp.float32),
        pltpu.SemaphoreType.DMA,
        pltpu.SemaphoreType.DMA,
    ],
    compiler_params=_SC_PARAMS,
)(_k3_body)

_k4 = pl.pallas_call(
    _k4_body,
    out_shape=jax.ShapeDtypeStruct((4, C), jnp.float32),
)


def kernel(fxyz, component):
    n = fxyz.shape[0]
    assert n % (NW * BLK) == 0
    xin = fxyz[:, 1]
    yin = fxyz[:, 2]
    zin = fxyz[:, 3]
    part = _k1(xin, yin, zin, component)
    ctr = _k2(part)
    dmaxp = _k3(xin, yin, zin, component, ctr)
    res = _k4(ctr, dmaxp)
    return res.T


# speedup vs baseline: 1.3308x; 1.3308x over previous
import dataclasses
import functools

import jax
import jax.numpy as jnp
from jax import lax
from jax.experimental import pallas as pl
from jax.experimental.pallas import tpu as pltpu
from jax.experimental.pallas import tpu_sc as plsc

C = 10000
NW = 32
L = 16
BLK = 2000
NV = 5
SS = NV * L

_MESH = plsc.VectorSubcoreMesh(
    core_axis_name="c", subcore_axis_name="s", num_cores=2, num_subcores=16)

_SC_PARAMS = pltpu.CompilerParams()
if "needs_layout_passes" in pltpu.CompilerParams.__dataclass_fields__:
    _SC_PARAMS = dataclasses.replace(_SC_PARAMS, needs_layout_passes=False)
if "use_tc_tiling_on_sc" in pltpu.CompilerParams.__dataclass_fields__:
    _SC_PARAMS = dataclasses.replace(_SC_PARAMS, use_tc_tiling_on_sc=False)


def _worker_id():
    return lax.axis_index("s") * 2 + lax.axis_index("c")


def _k1_body(x_hbm, y_hbm, z_hbm, comp_hbm, out_hbm,
             cnt_a, sx_a, sy_a, sz_a,
             xb, yb, zb, cbuf, xb2, yb2, zb2, cbuf2, sem0, sem1):
    n = comp_hbm.shape[0]
    chunk = n // NW
    nblk = chunk // BLK
    wid = _worker_id()
    base = wid * chunk
    lanes = lax.iota(jnp.int32, L)
    zeros = jnp.zeros((L,), jnp.float32)
    ones = jnp.ones((L,), jnp.float32)
    nvf = jnp.full((L,), float(NV), jnp.float32)
    bankoff = (lanes & 1) * C

    @pl.loop(0, 2 * C, step=L)
    def _zero(i):
        cnt_a[pl.ds(i, L)] = zeros
        sx_a[pl.ds(i, L)] = zeros
        sy_a[pl.ds(i, L)] = zeros
        sz_a[pl.ds(i, L)] = zeros

    lane0 = lanes == 0

    def flush(cur, vc, vx, vy, vz):
        bcur = jnp.broadcast_to(cur, (L,))
        plsc.addupdate_scatter(
            cnt_a, [bcur], jnp.broadcast_to(jnp.sum(vc), (L,)), mask=lane0)
        plsc.addupdate_scatter(
            sx_a, [bcur], jnp.broadcast_to(jnp.sum(vx), (L,)), mask=lane0)
        plsc.addupdate_scatter(
            sy_a, [bcur], jnp.broadcast_to(jnp.sum(vy), (L,)), mask=lane0)
        plsc.addupdate_scatter(
            sz_a, [bcur], jnp.broadcast_to(jnp.sum(vz), (L,)), mask=lane0)

    def dma4(bufs, sem, b, start):
        off = base + b * BLK
        cps = (
            pltpu.make_async_copy(x_hbm.at[pl.ds(off, BLK)], bufs[0], sem),
            pltpu.make_async_copy(y_hbm.at[pl.ds(off, BLK)], bufs[1], sem),
            pltpu.make_async_copy(z_hbm.at[pl.ds(off, BLK)], bufs[2], sem),
            pltpu.make_async_copy(comp_hbm.at[pl.ds(off, BLK)], bufs[3], sem),
        )
        for cp in cps:
            if start:
                cp.start()
            else:
                cp.wait()

    def block_compute(xb, yb, zb, cbuf):

        def step(t, carry):
            cur, vc, vx, vy, vz = carry
            i = t * SS
            xs = [xb[pl.ds(i + k * L, L)] for k in range(NV)]
            ys = [yb[pl.ds(i + k * L, L)] for k in range(NV)]
            zs = [zb[pl.ds(i + k * L, L)] for k in range(NV)]
            first = cbuf[pl.ds(i, L)]
            last = cbuf[pl.ds(i + SS - L, L)]
            c0 = first[0]
            cl = last[L - 1]

            def fast():
                sx = xs[0]
                sy = ys[0]
                sz = zs[0]
                for k in range(1, NV):
                    sx = sx + xs[k]
                    sy = sy + ys[k]
                    sz = sz + zs[k]
                return cur, vc + nvf, vx + sx, vy + sy, vz + sz

            def slow():
                flush(cur, vc, vx, vy, vz)
                for k in range(NV):
                    idx = cbuf[pl.ds(i + k * L, L)] + bankoff
                    plsc.addupdate_scatter(cnt_a, [idx], ones)
                    plsc.addupdate_scatter(sx_a, [idx], xs[k])
                    plsc.addupdate_scatter(sy_a, [idx], ys[k])
                    plsc.addupdate_scatter(sz_a, [idx], zs[k])
                return cl, zeros, zeros, zeros, zeros

            return lax.cond((c0 == cur) & (c0 == cl), fast, slow)

        init = (cbuf[pl.ds(0, L)][0], zeros, zeros, zeros, zeros)
        cur, vc, vx, vy, vz = lax.fori_loop(0, BLK // SS, step, init)
        flush(cur, vc, vx, vy, vz)

    b0 = (xb, yb, zb, cbuf)
    b1 = (xb2, yb2, zb2, cbuf2)
    dma4(b0, sem0, 0, True)

    @pl.loop(0, nblk // 2)
    def _pair(t):
        dma4(b0, sem0, 2 * t, False)
        dma4(b1, sem1, 2 * t + 1, True)
        block_compute(*b0)
        dma4(b1, sem1, 2 * t + 1, False)
        dma4(b0, sem0, 2 * t + 2, True)
        block_compute(*b1)

    dma4(b0, sem0, nblk - 1, False)
    block_compute(*b0)

    pltpu.sync_copy(cnt_a, out_hbm.at[wid, 0])
    pltpu.sync_copy(sx_a, out_hbm.at[wid, 1])
    pltpu.sync_copy(sy_a, out_hbm.at[wid, 2])
    pltpu.sync_copy(sz_a, out_hbm.at[wid, 3])


def _k2_body(part_ref, ctr_ref):
    s2 = jnp.sum(part_ref[...], axis=0)
    s = s2[:, :C] + s2[:, C:]
    deg = s[0:1, :]
    valid = deg > 0.5
    degs = jnp.where(valid, deg, 1.0)
    xyz = s[1:4, :]
    ctr = jnp.where(valid, xyz / degs, xyz)
    ctr_ref[...] = jnp.concatenate([ctr, deg], axis=0)


def _k3_body(x_hbm, y_hbm, z_hbm, comp_hbm, ctr_hbm, out_hbm,
             cxv, cyv, czv, dmx,
             xb, yb, zb, cbuf, xb2, yb2, zb2, cbuf2, sbuf, sem0, sem1):
    n = comp_hbm.shape[0]
    chunk = n // NW
    nblk = chunk // BLK
    wid = _worker_id()
    base = wid * chunk
    lanes = lax.iota(jnp.int32, L)
    lane0 = lanes == 0
    zeros = jnp.zeros((L,), jnp.float32)

    pltpu.sync_copy(ctr_hbm.at[0], cxv)
    pltpu.sync_copy(ctr_hbm.at[1], cyv)
    pltpu.sync_copy(ctr_hbm.at[2], czv)

    @pl.loop(0, C, step=L)
    def _zero(i):
        dmx[pl.ds(i, L)] = zeros

    cbuf[pl.ds(BLK, L)] = jnp.full((L,), -1, jnp.int32)
    cbuf2[pl.ds(BLK, L)] = jnp.full((L,), -1, jnp.int32)

    def dma4(bufs, sem, b, start):
        off = base + b * BLK
        cps = (
            pltpu.make_async_copy(x_hbm.at[pl.ds(off, BLK)], bufs[0], sem),
            pltpu.make_async_copy(y_hbm.at[pl.ds(off, BLK)], bufs[1], sem),
            pltpu.make_async_copy(z_hbm.at[pl.ds(off, BLK)], bufs[2], sem),
            pltpu.make_async_copy(
                comp_hbm.at[pl.ds(off, BLK)], bufs[3].at[pl.ds(0, BLK)], sem),
        )
        for cp in cps:
            if start:
                cp.start()
            else:
                cp.wait()

    def block_compute(xb, yb, zb, cbuf):

        def flush(cur, vmax):
            bcur = jnp.broadcast_to(cur, (L,))
            m = jnp.broadcast_to(jnp.max(vmax), (L,))
            old = plsc.load_gather(dmx, [bcur], mask=lane0)
            plsc.store_scatter(dmx, [bcur], jnp.maximum(old, m), mask=lane0)

        def centers(c):
            bc = jnp.broadcast_to(c, (L,))
            return (plsc.load_gather(cxv, [bc]),
                    plsc.load_gather(cyv, [bc]),
                    plsc.load_gather(czv, [bc]))

        def step(t, carry):
            cur, vmax, bcx, bcy, bcz = carry
            i = t * SS
            xs = [xb[pl.ds(i + k * L, L)] for k in range(NV)]
            ys = [yb[pl.ds(i + k * L, L)] for k in range(NV)]
            zs = [zb[pl.ds(i + k * L, L)] for k in range(NV)]
            first = cbuf[pl.ds(i, L)]
            last = cbuf[pl.ds(i + SS - L, L)]
            c0 = first[0]
            cl = last[L - 1]

            def fast():
                m = vmax
                for k in range(NV):
                    dx = xs[k] - bcx
                    dy = ys[k] - bcy
                    dz = zs[k] - bcz
                    m = jnp.maximum(m, dx * dx + dy * dy + dz * dz)
                return cur, m, bcx, bcy, bcz

            def slow():
                flush(cur, vmax)
                for k in range(NV):
                    ii = i + k * L
                    idx = cbuf[pl.ds(ii, L)]
                    cx = plsc.load_gather(cxv, [idx])
                    cy = plsc.load_gather(cyv, [idx])
                    cz = plsc.load_gather(czv, [idx])
                    dx = xs[k] - cx
                    dy = ys[k] - cy
                    dz = zs[k] - cz
                    v = dx * dx + dy * dy + dz * dz
                    for s in (1, 2, 4, 8):
                        sbuf[...] = v
                        vs = plsc.load_gather(
                            sbuf, [jnp.maximum(lanes - s, 0)])
                        ids = plsc.load_gather(
                            cbuf, [jnp.maximum(lanes + (ii - s), 0)])
                        keep = (lanes >= s) & (idx == ids)
                        v = jnp.where(keep, jnp.maximum(v, vs), v)
                    nxt = plsc.load_gather(cbuf, [lanes + (ii + 1)])
                    endm = (idx != nxt) | (lanes == L - 1)
                    old = plsc.load_gather(dmx, [idx], mask=endm)
                    plsc.store_scatter(
                        dmx, [idx], jnp.maximum(old, v), mask=endm)
                ncx, ncy, ncz = centers(cl)
                return cl, zeros, ncx, ncy, ncz

            return lax.cond((c0 == cur) & (c0 == cl), fast, slow)

        cc = cbuf[pl.ds(0, L)][0]
        icx, icy, icz = centers(cc)
        init = (cc, zeros, icx, icy, icz)
        cur, vmax, _, _, _ = lax.fori_loop(0, BLK // SS, step, init)
        flush(cur, vmax)

    b0 = (xb, yb, zb, cbuf)
    b1 = (xb2, yb2, zb2, cbuf2)
    dma4(b0, sem0, 0, True)

    @pl.loop(0, nblk // 2)
    def _pair(t):
        dma4(b0, sem0, 2 * t, False)
        dma4(b1, sem1, 2 * t + 1, True)
        block_compute(*b0)
        dma4(b1, sem1, 2 * t + 1, False)
        dma4(b0, sem0, 2 * t + 2, True)
        block_compute(*b1)

    dma4(b0, sem0, nblk - 1, False)
    block_compute(*b0)

    pltpu.sync_copy(dmx, out_hbm.at[wid])


def _k4_body(ctr_ref, dmx_ref, out_ref):
    m = jnp.max(dmx_ref[...], axis=0, keepdims=True)
    deg = ctr_ref[3:4, :]
    valid = deg > 0.5
    dia = jnp.where(valid, 2.0 * jnp.sqrt(m + 1e-12), 0.0)
    out_ref[...] = jnp.concatenate([ctr_ref[0:3, :], dia], axis=0)


_k1 = functools.partial(
    pl.kernel,
    out_type=jax.ShapeDtypeStruct((NW, 4, 2 * C), jnp.float32),
    mesh=_MESH,
    scratch_types=[
        pltpu.VMEM((2 * C,), jnp.float32),
        pltpu.VMEM((2 * C,), jnp.float32),
        pltpu.VMEM((2 * C,), jnp.float32),
        pltpu.VMEM((2 * C,), jnp.float32),
        pltpu.VMEM((BLK,), jnp.float32),
        pltpu.VMEM((BLK,), jnp.float32),
        pltpu.VMEM((BLK,), jnp.float32),
        pltpu.VMEM((BLK,), jnp.int32),
        pltpu.VMEM((BLK,), jnp.float32),
        pltpu.VMEM((BLK,), jnp.float32),
        pltpu.VMEM((BLK,), jnp.float32),
        pltpu.VMEM((BLK,), jnp.int32),
        pltpu.SemaphoreType.DMA,
        pltpu.SemaphoreType.DMA,
    ],
    compiler_params=_SC_PARAMS,
)(_k1_body)

_k2 = pl.pallas_call(
    _k2_body,
    out_shape=jax.ShapeDtypeStruct((4, C), jnp.float32),
)

_k3 = functools.partial(
    pl.kernel,
    out_type=jax.ShapeDtypeStruct((NW, C), jnp.float32),
    mesh=_MESH,
    scratch_types=[
        pltpu.VMEM((C,), jnp.float32),
        pltpu.VMEM((C,), jnp.float32),
        pltpu.VMEM((C,), jnp.float32),
        pltpu.VMEM((C,), jnp.float32),
        pltpu.VMEM((BLK,), jnp.float32),
        pltpu.VMEM((BLK,), jnp.float32),
        pltpu.VMEM((BLK,), jnp.float32),
        pltpu.VMEM((BLK + L,), jnp.int32),
        pltpu.VMEM((BLK,), jnp.float32),
        pltpu.VMEM((BLK,), jnp.float32),
        pltpu.VMEM((BLK,), jnp.float32),
        pltpu.VMEM((BLK + L,), jnp.int32),
        pltpu.VMEM((L,), jnp.float32),
        pltpu.SemaphoreType.DMA,
        pltpu.SemaphoreType.DMA,
    ],
    compiler_params=_SC_PARAMS,
)(_k3_body)

_k4 = pl.pallas_call(
    _k4_body,
    out_shape=jax.ShapeDtypeStruct((4, C), jnp.float32),
)


def kernel(fxyz, component):
    n = fxyz.shape[0]
    assert n % (NW * BLK) == 0
    xin = fxyz[:, 1]
    yin = fxyz[:, 2]
    zin = fxyz[:, 3]
    part = _k1(xin, yin, zin, component)
    ctr = _k2(part)
    dmaxp = _k3(xin, yin, zin, component, ctr)
    res = _k4(ctr, dmaxp)
    return res.T


# speedup vs baseline: 1.3755x; 1.0335x over previous
import dataclasses
import functools

import jax
import jax.numpy as jnp
from jax import lax
from jax.experimental import pallas as pl
from jax.experimental.pallas import tpu as pltpu
from jax.experimental.pallas import tpu_sc as plsc

C = 10000
NW = 32
L = 16
BLK = 2000
NV = 5
SS = NV * L

_MESH = plsc.VectorSubcoreMesh(
    core_axis_name="c", subcore_axis_name="s", num_cores=2, num_subcores=16)

_SC_PARAMS = pltpu.CompilerParams()
if "needs_layout_passes" in pltpu.CompilerParams.__dataclass_fields__:
    _SC_PARAMS = dataclasses.replace(_SC_PARAMS, needs_layout_passes=False)
if "use_tc_tiling_on_sc" in pltpu.CompilerParams.__dataclass_fields__:
    _SC_PARAMS = dataclasses.replace(_SC_PARAMS, use_tc_tiling_on_sc=False)


def _worker_id():
    return lax.axis_index("s") * 2 + lax.axis_index("c")


def _k1_body(x_hbm, y_hbm, z_hbm, comp_hbm, out_hbm,
             cnt_a, sx_a, sy_a, sz_a,
             xb, yb, zb, cbuf, xb2, yb2, zb2, cbuf2, sem0, sem1):
    n = comp_hbm.shape[0]
    chunk = n // NW
    nblk = chunk // BLK
    wid = _worker_id()
    base = wid * chunk
    lanes = lax.iota(jnp.int32, L)
    zeros = jnp.zeros((L,), jnp.float32)
    ones = jnp.ones((L,), jnp.float32)
    nvf = jnp.full((L,), float(NV), jnp.float32)
    bankoff = (lanes & 1) * C

    @pl.loop(0, 2 * C, step=L)
    def _zero(i):
        cnt_a[pl.ds(i, L)] = zeros
        sx_a[pl.ds(i, L)] = zeros
        sy_a[pl.ds(i, L)] = zeros
        sz_a[pl.ds(i, L)] = zeros

    lane0 = lanes == 0

    def flush(cur, vc, vx, vy, vz):
        bcur = jnp.broadcast_to(cur, (L,))
        plsc.addupdate_scatter(
            cnt_a, [bcur], jnp.broadcast_to(jnp.sum(vc), (L,)), mask=lane0)
        plsc.addupdate_scatter(
            sx_a, [bcur], jnp.broadcast_to(jnp.sum(vx), (L,)), mask=lane0)
        plsc.addupdate_scatter(
            sy_a, [bcur], jnp.broadcast_to(jnp.sum(vy), (L,)), mask=lane0)
        plsc.addupdate_scatter(
            sz_a, [bcur], jnp.broadcast_to(jnp.sum(vz), (L,)), mask=lane0)

    def dma4(bufs, sem, b, start):
        off = base + b * BLK
        cps = (
            pltpu.make_async_copy(x_hbm.at[pl.ds(off, BLK)], bufs[0], sem),
            pltpu.make_async_copy(y_hbm.at[pl.ds(off, BLK)], bufs[1], sem),
            pltpu.make_async_copy(z_hbm.at[pl.ds(off, BLK)], bufs[2], sem),
            pltpu.make_async_copy(comp_hbm.at[pl.ds(off, BLK)], bufs[3], sem),
        )
        for cp in cps:
            if start:
                cp.start()
            else:
                cp.wait()

    def block_compute(xb, yb, zb, cbuf):

        def step(t, carry):
            cur, vc, vx, vy, vz = carry
            i = t * SS
            xs = [xb[pl.ds(i + k * L, L)] for k in range(NV)]
            ys = [yb[pl.ds(i + k * L, L)] for k in range(NV)]
            zs = [zb[pl.ds(i + k * L, L)] for k in range(NV)]
            first = cbuf[pl.ds(i, L)]
            last = cbuf[pl.ds(i + SS - L, L)]
            c0 = first[0]
            cl = last[L - 1]

            def fast():
                sx = xs[0]
                sy = ys[0]
                sz = zs[0]
                for k in range(1, NV):
                    sx = sx + xs[k]
                    sy = sy + ys[k]
                    sz = sz + zs[k]
                return cur, vc + nvf, vx + sx, vy + sy, vz + sz

            def slow():
                flush(cur, vc, vx, vy, vz)
                for k in range(NV):
                    idx = cbuf[pl.ds(i + k * L, L)] + bankoff
                    plsc.addupdate_scatter(cnt_a, [idx], ones)
                    plsc.addupdate_scatter(sx_a, [idx], xs[k])
                    plsc.addupdate_scatter(sy_a, [idx], ys[k])
                    plsc.addupdate_scatter(sz_a, [idx], zs[k])
                return cl, zeros, zeros, zeros, zeros

            return lax.cond((c0 == cur) & (c0 == cl), fast, slow)

        init = (cbuf[pl.ds(0, L)][0], zeros, zeros, zeros, zeros)
        cur, vc, vx, vy, vz = lax.fori_loop(0, BLK // SS, step, init)
        flush(cur, vc, vx, vy, vz)

    b0 = (xb, yb, zb, cbuf)
    b1 = (xb2, yb2, zb2, cbuf2)
    dma4(b0, sem0, 0, True)

    @pl.loop(0, nblk // 2)
    def _pair(t):
        dma4(b0, sem0, 2 * t, False)
        dma4(b1, sem1, 2 * t + 1, True)
        block_compute(*b0)
        dma4(b1, sem1, 2 * t + 1, False)
        dma4(b0, sem0, 2 * t + 2, True)
        block_compute(*b1)

    dma4(b0, sem0, nblk - 1, False)
    block_compute(*b0)

    @pl.loop(0, C, step=L)
    def _fold(i):
        cnt_a[pl.ds(i, L)] = cnt_a[pl.ds(i, L)] + cnt_a[pl.ds(C + i, L)]
        sx_a[pl.ds(i, L)] = sx_a[pl.ds(i, L)] + sx_a[pl.ds(C + i, L)]
        sy_a[pl.ds(i, L)] = sy_a[pl.ds(i, L)] + sy_a[pl.ds(C + i, L)]
        sz_a[pl.ds(i, L)] = sz_a[pl.ds(i, L)] + sz_a[pl.ds(C + i, L)]

    pltpu.sync_copy(cnt_a.at[pl.ds(0, C)], out_hbm.at[wid, 0])
    pltpu.sync_copy(sx_a.at[pl.ds(0, C)], out_hbm.at[wid, 1])
    pltpu.sync_copy(sy_a.at[pl.ds(0, C)], out_hbm.at[wid, 2])
    pltpu.sync_copy(sz_a.at[pl.ds(0, C)], out_hbm.at[wid, 3])


def _k2_body(part_ref, ctr_ref):
    s = jnp.sum(part_ref[...], axis=0)
    deg = s[0:1, :]
    valid = deg > 0.5
    degs = jnp.where(valid, deg, 1.0)
    xyz = s[1:4, :]
    ctr = jnp.where(valid, xyz / degs, xyz)
    ctr_ref[...] = jnp.concatenate([ctr, deg], axis=0)


def _k3_body(x_hbm, y_hbm, z_hbm, comp_hbm, ctr_hbm, out_hbm,
             cxv, cyv, czv, dmx,
             xb, yb, zb, cbuf, xb2, yb2, zb2, cbuf2, sbuf, sem0, sem1):
    n = comp_hbm.shape[0]
    chunk = n // NW
    nblk = chunk // BLK
    wid = _worker_id()
    base = wid * chunk
    lanes = lax.iota(jnp.int32, L)
    lane0 = lanes == 0
    zeros = jnp.zeros((L,), jnp.float32)

    pltpu.sync_copy(ctr_hbm.at[0], cxv)
    pltpu.sync_copy(ctr_hbm.at[1], cyv)
    pltpu.sync_copy(ctr_hbm.at[2], czv)

    @pl.loop(0, C, step=L)
    def _zero(i):
        dmx[pl.ds(i, L)] = zeros

    cbuf[pl.ds(BLK, L)] = jnp.full((L,), -1, jnp.int32)
    cbuf2[pl.ds(BLK, L)] = jnp.full((L,), -1, jnp.int32)

    def dma4(bufs, sem, b, start):
        off = base + b * BLK
        cps = (
            pltpu.make_async_copy(x_hbm.at[pl.ds(off, BLK)], bufs[0], sem),
            pltpu.make_async_copy(y_hbm.at[pl.ds(off, BLK)], bufs[1], sem),
            pltpu.make_async_copy(z_hbm.at[pl.ds(off, BLK)], bufs[2], sem),
            pltpu.make_async_copy(
                comp_hbm.at[pl.ds(off, BLK)], bufs[3].at[pl.ds(0, BLK)], sem),
        )
        for cp in cps:
            if start:
                cp.start()
            else:
                cp.wait()

    def block_compute(xb, yb, zb, cbuf):

        def flush(cur, vmax):
            bcur = jnp.broadcast_to(cur, (L,))
            m = jnp.broadcast_to(jnp.max(vmax), (L,))
            old = plsc.load_gather(dmx, [bcur], mask=lane0)
            plsc.store_scatter(dmx, [bcur], jnp.maximum(old, m), mask=lane0)

        def centers(c):
            bc = jnp.broadcast_to(c, (L,))
            return (plsc.load_gather(cxv, [bc]),
                    plsc.load_gather(cyv, [bc]),
                    plsc.load_gather(czv, [bc]))

        def step(t, carry):
            cur, vmax, bcx, bcy, bcz = carry
            i = t * SS
            xs = [xb[pl.ds(i + k * L, L)] for k in range(NV)]
            ys = [yb[pl.ds(i + k * L, L)] for k in range(NV)]
            zs = [zb[pl.ds(i + k * L, L)] for k in range(NV)]
            first = cbuf[pl.ds(i, L)]
            last = cbuf[pl.ds(i + SS - L, L)]
            c0 = first[0]
            cl = last[L - 1]

            def fast():
                m = vmax
                for k in range(NV):
                    dx = xs[k] - bcx
                    dy = ys[k] - bcy
                    dz = zs[k] - bcz
                    m = jnp.maximum(m, dx * dx + dy * dy + dz * dz)
                return cur, m, bcx, bcy, bcz

            def slow():
                flush(cur, vmax)
                for k in range(NV):
                    ii = i + k * L
                    idx = cbuf[pl.ds(ii, L)]
                    cx = plsc.load_gather(cxv, [idx])
                    cy = plsc.load_gather(cyv, [idx])
                    cz = plsc.load_gather(czv, [idx])
                    dx = xs[k] - cx
                    dy = ys[k] - cy
                    dz = zs[k] - cz
                    v = dx * dx + dy * dy + dz * dz
                    for s in (1, 2, 4, 8):
                        sbuf[...] = v
                        vs = plsc.load_gather(
                            sbuf, [jnp.maximum(lanes - s, 0)])
                        ids = plsc.load_gather(
                            cbuf, [jnp.maximum(lanes + (ii - s), 0)])
                        keep = (lanes >= s) & (idx == ids)
                        v = jnp.where(keep, jnp.maximum(v, vs), v)
                    nxt = plsc.load_gather(cbuf, [lanes + (ii + 1)])
                    endm = (idx != nxt) | (lanes == L - 1)
                    old = plsc.load_gather(dmx, [idx], mask=endm)
                    plsc.store_scatter(
                        dmx, [idx], jnp.maximum(old, v), mask=endm)
                ncx, ncy, ncz = centers(cl)
                return cl, zeros, ncx, ncy, ncz

            return lax.cond((c0 == cur) & (c0 == cl), fast, slow)

        cc = cbuf[pl.ds(0, L)][0]
        icx, icy, icz = centers(cc)
        init = (cc, zeros, icx, icy, icz)
        cur, vmax, _, _, _ = lax.fori_loop(0, BLK // SS, step, init)
        flush(cur, vmax)

    b0 = (xb, yb, zb, cbuf)
    b1 = (xb2, yb2, zb2, cbuf2)
    dma4(b0, sem0, 0, True)

    @pl.loop(0, nblk // 2)
    def _pair(t):
        dma4(b0, sem0, 2 * t, False)
        dma4(b1, sem1, 2 * t + 1, True)
        block_compute(*b0)
        dma4(b1, sem1, 2 * t + 1, False)
        dma4(b0, sem0, 2 * t + 2, True)
        block_compute(*b1)

    dma4(b0, sem0, nblk - 1, False)
    block_compute(*b0)

    pltpu.sync_copy(dmx, out_hbm.at[wid])


def _k4_body(ctr_ref, dmx_ref, out_ref):
    m = jnp.max(dmx_ref[...], axis=0, keepdims=True)
    deg = ctr_ref[3:4, :]
    valid = deg > 0.5
    dia = jnp.where(valid, 2.0 * jnp.sqrt(m + 1e-12), 0.0)
    out_ref[...] = jnp.concatenate([ctr_ref[0:3, :], dia], axis=0)


_k1 = functools.partial(
    pl.kernel,
    out_type=jax.ShapeDtypeStruct((NW, 4, C), jnp.float32),
    mesh=_MESH,
    scratch_types=[
        pltpu.VMEM((2 * C,), jnp.float32),
        pltpu.VMEM((2 * C,), jnp.float32),
        pltpu.VMEM((2 * C,), jnp.float32),
        pltpu.VMEM((2 * C,), jnp.float32),
        pltpu.VMEM((BLK,), jnp.float32),
        pltpu.VMEM((BLK,), jnp.float32),
        pltpu.VMEM((BLK,), jnp.float32),
        pltpu.VMEM((BLK,), jnp.int32),
        pltpu.VMEM((BLK,), jnp.float32),
        pltpu.VMEM((BLK,), jnp.float32),
        pltpu.VMEM((BLK,), jnp.float32),
        pltpu.VMEM((BLK,), jnp.int32),
        pltpu.SemaphoreType.DMA,
        pltpu.SemaphoreType.DMA,
    ],
    compiler_params=_SC_PARAMS,
)(_k1_body)

_k2 = pl.pallas_call(
    _k2_body,
    out_shape=jax.ShapeDtypeStruct((4, C), jnp.float32),
)

_k3 = functools.partial(
    pl.kernel,
    out_type=jax.ShapeDtypeStruct((NW, C), jnp.float32),
    mesh=_MESH,
    scratch_types=[
        pltpu.VMEM((C,), jnp.float32),
        pltpu.VMEM((C,), jnp.float32),
        pltpu.VMEM((C,), jnp.float32),
        pltpu.VMEM((C,), jnp.float32),
        pltpu.VMEM((BLK,), jnp.float32),
        pltpu.VMEM((BLK,), jnp.float32),
        pltpu.VMEM((BLK,), jnp.float32),
        pltpu.VMEM((BLK + L,), jnp.int32),
        pltpu.VMEM((BLK,), jnp.float32),
        pltpu.VMEM((BLK,), jnp.float32),
        pltpu.VMEM((BLK,), jnp.float32),
        pltpu.VMEM((BLK + L,), jnp.int32),
        pltpu.VMEM((L,), jnp.float32),
        pltpu.SemaphoreType.DMA,
        pltpu.SemaphoreType.DMA,
    ],
    compiler_params=_SC_PARAMS,
)(_k3_body)

_k4 = pl.pallas_call(
    _k4_body,
    out_shape=jax.ShapeDtypeStruct((4, C), jnp.float32),
)


def kernel(fxyz, component):
    n = fxyz.shape[0]
    assert n % (NW * BLK) == 0
    xin = fxyz[:, 1]
    yin = fxyz[:, 2]
    zin = fxyz[:, 3]
    part = _k1(xin, yin, zin, component)
    ctr = _k2(part)
    dmaxp = _k3(xin, yin, zin, component, ctr)
    res = _k4(ctr, dmaxp)
    return res.T
